# Initial kernel scaffold; baseline (speedup 1.0000x reference)
import functools

import jax
import jax.numpy as jnp
from jax import lax
from jax.experimental import pallas as pl
from jax.experimental.pallas import tpu as pltpu
from jax.experimental.pallas import tpu_sc as plsc

NC = 2
NS = 16
CH = 128
DEG_W = 16


def _make_prop(NP, EPW, H, with_deg):
    n_chunks = EPW // CH
    rpt = NP // NS
    nz = rpt // CH
    mesh = plsc.VectorSubcoreMesh(
        core_axis_name="c", subcore_axis_name="s",
        num_cores=NC, num_subcores=NS)
    out_type = [jax.ShapeDtypeStruct((NC, NP, H), jnp.float32)]
    scratch = [
        pltpu.VMEM((CH, H), jnp.float32),
        pltpu.VMEM((CH,), jnp.int32),
        pltpu.VMEM((CH,), jnp.int32),
        pltpu.VMEM_SHARED((NP, H), jnp.float32),
    ]
    if with_deg:
        out_type.append(jax.ShapeDtypeStruct((NC, NP, DEG_W), jnp.float32))
        scratch += [
            pltpu.VMEM((CH, DEG_W), jnp.float32),
            pltpu.VMEM((CH, DEG_W), jnp.float32),
            pltpu.VMEM_SHARED((NP, DEG_W), jnp.float32),
        ]

    def body(*refs):
        if with_deg:
            (h_hbm, col_hbm, row_hbm, out_hbm, dout_hbm,
             rows_v, col_v, row_v, acc, ones_v, zb_v, dacc) = refs
        else:
            (h_hbm, col_hbm, row_hbm, out_hbm,
             rows_v, col_v, row_v, acc) = refs
        cid = lax.axis_index("c")
        sid = lax.axis_index("s")
        wid = sid * NC + cid
        zero16 = jnp.zeros((16,), jnp.float32)
        one16 = jnp.ones((16,), jnp.float32)

        @pl.loop(0, CH)
        def _zrows(i):
            @pl.loop(0, H // 16)
            def _zlanes(j):
                rows_v[i, pl.ds(j * 16, 16)] = zero16

        if with_deg:
            @pl.loop(0, CH)
            def _fill(i):
                ones_v[i, pl.ds(0, 16)] = one16
                zb_v[i, pl.ds(0, 16)] = zero16

        row0 = sid * rpt

        @pl.loop(0, nz)
        def _zacc(i):
            pltpu.sync_copy(rows_v, acc.at[pl.ds(row0 + i * CH, CH)])
            if with_deg:
                pltpu.sync_copy(zb_v, dacc.at[pl.ds(row0 + i * CH, CH)])

        plsc.subcore_barrier()

        base0 = wid * EPW

        @pl.loop(0, n_chunks)
        def _edges(i):
            base = base0 + i * CH
            pltpu.sync_copy(col_hbm.at[pl.ds(base, CH)], col_v)
            pltpu.sync_copy(row_hbm.at[pl.ds(base, CH)], row_v)
            pltpu.sync_copy(h_hbm.at[col_v], rows_v)
            pltpu.sync_copy(rows_v, acc.at[row_v], add=True)
            if with_deg:
                pltpu.sync_copy(ones_v, dacc.at[row_v], add=True)

        plsc.subcore_barrier()

        pltpu.sync_copy(acc.at[pl.ds(row0, rpt)],
                        out_hbm.at[cid, pl.ds(row0, rpt)])
        if with_deg:
            pltpu.sync_copy(dacc.at[pl.ds(row0, rpt)],
                            dout_hbm.at[cid, pl.ds(row0, rpt)])

    return pl.kernel(body, out_type=tuple(out_type), mesh=mesh,
                     scratch_types=tuple(scratch))


def _lin_in(x_pad, WinT, b_in2, NP, D, H, BN):
    def body(x_ref, w_ref, b_ref, o_ref):
        h = jnp.dot(x_ref[...], w_ref[...],
                    preferred_element_type=jnp.float32,
                    precision=lax.Precision.HIGHEST)
        o_ref[...] = jnp.maximum(h + b_ref[...], 0.0)

    return pl.pallas_call(
        body,
        grid=(NP // BN,),
        in_specs=[pl.BlockSpec((BN, D), lambda i: (i, 0)),
                  pl.BlockSpec((D, H), lambda i: (0, 0)),
                  pl.BlockSpec((1, H), lambda i: (0, 0))],
        out_specs=pl.BlockSpec((BN, H), lambda i: (i, 0)),
        out_shape=jax.ShapeDtypeStruct((NP, H), jnp.float32),
    )(x_pad, WinT, b_in2)


def _combine_deg(P0, P1, D0, D1, NP, H, BN):
    def body(p0, p1, d0, d1, m_ref, inv_ref):
        deg = d0[...] + d1[...]
        inv = jnp.where(deg > 0.0, 1.0 / deg, 0.0)
        inv_ref[...] = inv
        m_ref[...] = (p0[...] + p1[...]) * inv[:, 0:1]

    return pl.pallas_call(
        body,
        grid=(NP // BN,),
        in_specs=[pl.BlockSpec((BN, H), lambda i: (i, 0)),
                  pl.BlockSpec((BN, H), lambda i: (i, 0)),
                  pl.BlockSpec((BN, DEG_W), lambda i: (i, 0)),
                  pl.BlockSpec((BN, DEG_W), lambda i: (i, 0))],
        out_specs=[pl.BlockSpec((BN, H), lambda i: (i, 0)),
                   pl.BlockSpec((BN, DEG_W), lambda i: (i, 0))],
        out_shape=[jax.ShapeDtypeStruct((NP, H), jnp.float32),
                   jax.ShapeDtypeStruct((NP, DEG_W), jnp.float32)],
    )(P0, P1, D0, D1)


def _final(h, m1, Q0, Q1, inv, WcT, b_out2, NP, H, O, BN):
    def body(h_ref, m1_ref, q0, q1, inv_ref, w_ref, b_ref, o_ref):
        m2 = (q0[...] + q1[...]) * inv_ref[...][:, 0:1]
        comb = jnp.concatenate([h_ref[...], m1_ref[...], m2], axis=1)
        o_ref[...] = jnp.dot(comb, w_ref[...],
                             preferred_element_type=jnp.float32,
                             precision=lax.Precision.HIGHEST) + b_ref[...]

    return pl.pallas_call(
        body,
        grid=(NP // BN,),
        in_specs=[pl.BlockSpec((BN, H), lambda i: (i, 0)),
                  pl.BlockSpec((BN, H), lambda i: (i, 0)),
                  pl.BlockSpec((BN, H), lambda i: (i, 0)),
                  pl.BlockSpec((BN, H), lambda i: (i, 0)),
                  pl.BlockSpec((BN, DEG_W), lambda i: (i, 0)),
                  pl.BlockSpec((3 * H, O), lambda i: (0, 0)),
                  pl.BlockSpec((1, O), lambda i: (0, 0))],
        out_specs=pl.BlockSpec((BN, O), lambda i: (i, 0)),
        out_shape=jax.ShapeDtypeStruct((NP, O), jnp.float32),
    )(h, m1, Q0, Q1, inv, WcT, b_out2)


def kernel(x, edge_index, W_in, b_in, W_out, b_out):
    N, D = x.shape
    H = W_in.shape[0]
    O = W_out.shape[0]
    E = edge_index.shape[1]

    NP = -(-(N + 1) // 2048) * 2048
    EPW = -(-E // (NC * NS * CH)) * CH
    Epad = EPW * NC * NS
    BN = 1024

    row = edge_index[0].astype(jnp.int32)
    col = edge_index[1].astype(jnp.int32)
    row = jnp.concatenate([row, jnp.full((Epad - E,), N, jnp.int32)])
    col = jnp.concatenate([col, jnp.zeros((Epad - E,), jnp.int32)])

    x_pad = jnp.pad(x, ((0, NP - N), (0, 0)))
    WinT = W_in.T
    b_in2 = b_in.reshape(1, H)
    W0 = W_out[:, 0:H]
    W12 = W_out[:, H:2 * H] + W_out[:, 2 * H:3 * H]
    W34 = W_out[:, 3 * H:4 * H] + W_out[:, 4 * H:5 * H]
    WcT = jnp.concatenate([W0, W12, W34], axis=1).T
    b_out2 = b_out.reshape(1, O)

    h = _lin_in(x_pad, WinT, b_in2, NP, D, H, BN)

    P, Dg = _make_prop(NP, EPW, H, True)(h, col, row)
    m1, inv = _combine_deg(P[0], P[1], Dg[0], Dg[1], NP, H, BN)

    Q = _make_prop(NP, EPW, H, False)(m1, col, row)
    if isinstance(Q, (list, tuple)):
        Q = Q[0]

    out = _final(h, m1, Q[0], Q[1], inv, WcT, b_out2, NP, H, O, BN)
    return out[:N]


# speedup vs baseline: 4.9697x; 4.9697x over previous
import functools

import jax
import jax.numpy as jnp
from jax import lax
from jax.experimental import pallas as pl
from jax.experimental.pallas import tpu as pltpu
from jax.experimental.pallas import tpu_sc as plsc

NC = 2
NS = 16
CH = 128
DEG_W = 16


def _make_prop(NP, EPW, H, with_deg):
    n_chunks = EPW // CH
    rpt = NP // NS
    nz = rpt // CH
    mesh = plsc.VectorSubcoreMesh(
        core_axis_name="c", subcore_axis_name="s",
        num_cores=NC, num_subcores=NS)
    out_type = [jax.ShapeDtypeStruct((NC, NP, H), jnp.float32)]
    scratch = [
        pltpu.VMEM((CH, H), jnp.float32),
        pltpu.VMEM((CH,), jnp.int32),
        pltpu.VMEM((CH,), jnp.int32),
        pltpu.VMEM_SHARED((NP, H), jnp.float32),
    ]
    if with_deg:
        out_type.append(jax.ShapeDtypeStruct((NC, NP, DEG_W), jnp.float32))
        scratch += [
            pltpu.VMEM((CH, DEG_W), jnp.float32),
            pltpu.VMEM((CH, DEG_W), jnp.float32),
            pltpu.VMEM_SHARED((NP, DEG_W), jnp.float32),
        ]

    def body(*refs):
        if with_deg:
            (h_hbm, col_hbm, row_hbm, out_hbm, dout_hbm,
             rows_v, col_v, row_v, acc, ones_v, zb_v, dacc) = refs
        else:
            (h_hbm, col_hbm, row_hbm, out_hbm,
             rows_v, col_v, row_v, acc) = refs
        cid = lax.axis_index("c")
        sid = lax.axis_index("s")
        wid = sid * NC + cid
        zero16 = jnp.zeros((16,), jnp.float32)
        one16 = jnp.ones((16,), jnp.float32)

        @pl.loop(0, CH)
        def _zrows(i):
            @pl.loop(0, H // 16)
            def _zlanes(j):
                rows_v[i, pl.ds(j * 16, 16)] = zero16

        if with_deg:
            @pl.loop(0, CH)
            def _fill(i):
                ones_v[i, pl.ds(0, 16)] = one16
                zb_v[i, pl.ds(0, 16)] = zero16

        row0 = sid * rpt

        @pl.loop(0, nz)
        def _zacc(i):
            pltpu.sync_copy(rows_v, acc.at[pl.ds(row0 + i * CH, CH)])
            if with_deg:
                pltpu.sync_copy(zb_v, dacc.at[pl.ds(row0 + i * CH, CH)])

        plsc.subcore_barrier()

        base0 = wid * EPW

        @pl.loop(0, n_chunks)
        def _edges(i):
            base = base0 + i * CH
            pltpu.sync_copy(col_hbm.at[pl.ds(base, CH)], col_v)
            pltpu.sync_copy(row_hbm.at[pl.ds(base, CH)], row_v)
            pltpu.sync_copy(h_hbm.at[col_v], rows_v)
            pltpu.sync_copy(rows_v, acc.at[row_v], add=True)
            if with_deg:
                pltpu.sync_copy(ones_v, dacc.at[row_v], add=True)

        plsc.subcore_barrier()

        pltpu.sync_copy(acc.at[pl.ds(row0, rpt)],
                        out_hbm.at[cid, pl.ds(row0, rpt)])
        if with_deg:
            pltpu.sync_copy(dacc.at[pl.ds(row0, rpt)],
                            dout_hbm.at[cid, pl.ds(row0, rpt)])

    return pl.kernel(body, out_type=tuple(out_type), mesh=mesh,
                     scratch_types=tuple(scratch),
                     compiler_params=pltpu.CompilerParams(
                         use_tc_tiling_on_sc=False))


def _lin_in(x_pad, WinT, b_in2, NP, D, H, BN):
    def body(x_ref, w_ref, b_ref, o_ref):
        h = jnp.dot(x_ref[...], w_ref[...],
                    preferred_element_type=jnp.float32,
                    precision=lax.Precision.HIGHEST)
        o_ref[...] = jnp.maximum(h + b_ref[...], 0.0)

    return pl.pallas_call(
        body,
        grid=(NP // BN,),
        in_specs=[pl.BlockSpec((BN, D), lambda i: (i, 0)),
                  pl.BlockSpec((D, H), lambda i: (0, 0)),
                  pl.BlockSpec((1, H), lambda i: (0, 0))],
        out_specs=pl.BlockSpec((BN, H), lambda i: (i, 0)),
        out_shape=jax.ShapeDtypeStruct((NP, H), jnp.float32),
    )(x_pad, WinT, b_in2)


def _combine_deg(P0, P1, D0, D1, NP, H, BN):
    def body(p0, p1, d0, d1, m_ref, inv_ref):
        deg = d0[...] + d1[...]
        inv = jnp.where(deg > 0.0, 1.0 / deg, 0.0)
        inv_ref[...] = inv
        m_ref[...] = (p0[...] + p1[...]) * inv[:, 0:1]

    return pl.pallas_call(
        body,
        grid=(NP // BN,),
        in_specs=[pl.BlockSpec((BN, H), lambda i: (i, 0)),
                  pl.BlockSpec((BN, H), lambda i: (i, 0)),
                  pl.BlockSpec((BN, DEG_W), lambda i: (i, 0)),
                  pl.BlockSpec((BN, DEG_W), lambda i: (i, 0))],
        out_specs=[pl.BlockSpec((BN, H), lambda i: (i, 0)),
                   pl.BlockSpec((BN, DEG_W), lambda i: (i, 0))],
        out_shape=[jax.ShapeDtypeStruct((NP, H), jnp.float32),
                   jax.ShapeDtypeStruct((NP, DEG_W), jnp.float32)],
    )(P0, P1, D0, D1)


def _final(h, m1, Q0, Q1, inv, WcT, b_out2, NP, H, O, BN):
    def body(h_ref, m1_ref, q0, q1, inv_ref, w_ref, b_ref, o_ref):
        m2 = (q0[...] + q1[...]) * inv_ref[...][:, 0:1]
        comb = jnp.concatenate([h_ref[...], m1_ref[...], m2], axis=1)
        o_ref[...] = jnp.dot(comb, w_ref[...],
                             preferred_element_type=jnp.float32,
                             precision=lax.Precision.HIGHEST) + b_ref[...]

    return pl.pallas_call(
        body,
        grid=(NP // BN,),
        in_specs=[pl.BlockSpec((BN, H), lambda i: (i, 0)),
                  pl.BlockSpec((BN, H), lambda i: (i, 0)),
                  pl.BlockSpec((BN, H), lambda i: (i, 0)),
                  pl.BlockSpec((BN, H), lambda i: (i, 0)),
                  pl.BlockSpec((BN, DEG_W), lambda i: (i, 0)),
                  pl.BlockSpec((3 * H, O), lambda i: (0, 0)),
                  pl.BlockSpec((1, O), lambda i: (0, 0))],
        out_specs=pl.BlockSpec((BN, O), lambda i: (i, 0)),
        out_shape=jax.ShapeDtypeStruct((NP, O), jnp.float32),
    )(h, m1, Q0, Q1, inv, WcT, b_out2)


def kernel(x, edge_index, W_in, b_in, W_out, b_out):
    N, D = x.shape
    H = W_in.shape[0]
    O = W_out.shape[0]
    E = edge_index.shape[1]

    NP = -(-(N + 1) // 2048) * 2048
    EPW = -(-E // (NC * NS * CH)) * CH
    Epad = EPW * NC * NS
    BN = 1024

    row = edge_index[0].astype(jnp.int32)
    col = edge_index[1].astype(jnp.int32)
    row = jnp.concatenate([row, jnp.full((Epad - E,), N, jnp.int32)])
    col = jnp.concatenate([col, jnp.zeros((Epad - E,), jnp.int32)])

    x_pad = jnp.pad(x, ((0, NP - N), (0, 0)))
    WinT = W_in.T
    b_in2 = b_in.reshape(1, H)
    W0 = W_out[:, 0:H]
    W12 = W_out[:, H:2 * H] + W_out[:, 2 * H:3 * H]
    W34 = W_out[:, 3 * H:4 * H] + W_out[:, 4 * H:5 * H]
    WcT = jnp.concatenate([W0, W12, W34], axis=1).T
    b_out2 = b_out.reshape(1, O)

    h = _lin_in(x_pad, WinT, b_in2, NP, D, H, BN)

    P, Dg = _make_prop(NP, EPW, H, True)(h, col, row)
    m1, inv = _combine_deg(P[0], P[1], Dg[0], Dg[1], NP, H, BN)

    Q = _make_prop(NP, EPW, H, False)(m1, col, row)
    if isinstance(Q, (list, tuple)):
        Q = Q[0]

    out = _final(h, m1, Q[0], Q[1], inv, WcT, b_out2, NP, H, O, BN)
    return out[:N]


# speedup vs baseline: 5.2095x; 1.0483x over previous
import functools

import jax
import jax.numpy as jnp
from jax import lax
from jax.experimental import pallas as pl
from jax.experimental.pallas import tpu as pltpu
from jax.experimental.pallas import tpu_sc as plsc

NC = 2
NS = 16
CH = 128
DEG_W = 16


NBUF = 4


def _make_prop(NP, EPW, H, with_deg):
    npc = EPW // CH
    n_groups = npc // NBUF
    rpt = NP // NS
    nz = rpt // CH
    mesh = plsc.VectorSubcoreMesh(
        core_axis_name="c", subcore_axis_name="s",
        num_cores=NC, num_subcores=NS)
    out_type = [jax.ShapeDtypeStruct((NC, NP, H), jnp.float32)]
    scratch = [
        pltpu.VMEM((npc, CH), jnp.int32),
        pltpu.VMEM((npc, CH), jnp.int32),
        pltpu.VMEM_SHARED((NP, H), jnp.float32),
    ]
    scratch += [pltpu.VMEM((CH, H), jnp.float32) for _ in range(NBUF)]
    scratch += [pltpu.SemaphoreType.DMA for _ in range(2 * NBUF)]
    if with_deg:
        out_type.append(jax.ShapeDtypeStruct((NC, NP, DEG_W), jnp.float32))
        scratch += [
            pltpu.VMEM((CH, DEG_W), jnp.float32),
            pltpu.VMEM((CH, DEG_W), jnp.float32),
            pltpu.VMEM_SHARED((NP, DEG_W), jnp.float32),
        ]
        scratch += [pltpu.SemaphoreType.DMA for _ in range(NBUF)]

    def body(*refs):
        h_hbm, col_hbm, row_hbm = refs[0:3]
        if with_deg:
            out_hbm, dout_hbm = refs[3:5]
            rest = refs[5:]
        else:
            out_hbm = refs[3]
            rest = refs[4:]
        col_s, row_s, acc = rest[0:3]
        rows_v = rest[3:3 + NBUF]
        gsem = rest[3 + NBUF:3 + 2 * NBUF]
        ssem = rest[3 + 2 * NBUF:3 + 3 * NBUF]
        if with_deg:
            ones_v, zb_v, dacc = rest[3 + 3 * NBUF:6 + 3 * NBUF]
            dsem = rest[6 + 3 * NBUF:6 + 4 * NBUF]

        cid = lax.axis_index("c")
        sid = lax.axis_index("s")
        wid = sid * NC + cid
        zero16 = jnp.zeros((16,), jnp.float32)
        one16 = jnp.ones((16,), jnp.float32)

        idesc0 = pltpu.async_copy(col_hbm.at[pl.ds(wid * npc, npc)], col_s,
                                  gsem[0])
        idesc1 = pltpu.async_copy(row_hbm.at[pl.ds(wid * npc, npc)], row_s,
                                  gsem[1])

        @pl.loop(0, CH)
        def _zrows(i):
            @pl.loop(0, H // 16)
            def _zlanes(j):
                rows_v[0][i, pl.ds(j * 16, 16)] = zero16

        if with_deg:
            @pl.loop(0, CH)
            def _fill(i):
                ones_v[i, pl.ds(0, 16)] = one16
                zb_v[i, pl.ds(0, 16)] = zero16

        row0 = sid * rpt

        @pl.loop(0, nz)
        def _zacc(i):
            pltpu.sync_copy(rows_v[0], acc.at[pl.ds(row0 + i * CH, CH)])
            if with_deg:
                pltpu.sync_copy(zb_v, dacc.at[pl.ds(row0 + i * CH, CH)])

        idesc0.wait()
        idesc1.wait()
        for b in range(NBUF):
            pltpu.async_copy(h_hbm.at[col_s.at[b]], rows_v[b], gsem[b])

        plsc.subcore_barrier()

        @pl.loop(0, n_groups)
        def _edges(g):
            c0 = g * NBUF
            for b in range(NBUF):
                c = c0 + b
                pltpu.make_async_copy(h_hbm.at[col_s.at[c]], rows_v[b],
                                      gsem[b]).wait()
                pltpu.async_copy(rows_v[b], acc.at[row_s.at[c]], ssem[b],
                                 add=True)
                if with_deg:
                    pltpu.async_copy(ones_v, dacc.at[row_s.at[c]], dsem[b],
                                     add=True)
            for b in range(NBUF):
                c = c0 + b
                pltpu.make_async_copy(rows_v[b], acc.at[row_s.at[c]],
                                      ssem[b]).wait()
                if with_deg:
                    pltpu.make_async_copy(ones_v, dacc.at[row_s.at[c]],
                                          dsem[b]).wait()

                @pl.when(g + 1 < n_groups)
                def _refill():
                    pltpu.async_copy(h_hbm.at[col_s.at[c + NBUF]], rows_v[b],
                                     gsem[b])

        plsc.subcore_barrier()

        pltpu.sync_copy(acc.at[pl.ds(row0, rpt)],
                        out_hbm.at[cid, pl.ds(row0, rpt)])
        if with_deg:
            pltpu.sync_copy(dacc.at[pl.ds(row0, rpt)],
                            dout_hbm.at[cid, pl.ds(row0, rpt)])

    return pl.kernel(body, out_type=tuple(out_type), mesh=mesh,
                     scratch_types=tuple(scratch),
                     compiler_params=pltpu.CompilerParams(
                         use_tc_tiling_on_sc=False))


def _lin_in(x_pad, WinT, b_in2, NP, D, H, BN):
    def body(x_ref, w_ref, b_ref, o_ref):
        h = jnp.dot(x_ref[...], w_ref[...],
                    preferred_element_type=jnp.float32,
                    precision=lax.Precision.HIGHEST)
        o_ref[...] = jnp.maximum(h + b_ref[...], 0.0)

    return pl.pallas_call(
        body,
        grid=(NP // BN,),
        in_specs=[pl.BlockSpec((BN, D), lambda i: (i, 0)),
                  pl.BlockSpec((D, H), lambda i: (0, 0)),
                  pl.BlockSpec((1, H), lambda i: (0, 0))],
        out_specs=pl.BlockSpec((BN, H), lambda i: (i, 0)),
        out_shape=jax.ShapeDtypeStruct((NP, H), jnp.float32),
    )(x_pad, WinT, b_in2)


def _combine_deg(P0, P1, D0, D1, NP, H, BN):
    def body(p0, p1, d0, d1, m_ref, inv_ref):
        deg = d0[...] + d1[...]
        inv = jnp.where(deg > 0.0, 1.0 / deg, 0.0)
        inv_ref[...] = inv
        m_ref[...] = (p0[...] + p1[...]) * inv[:, 0:1]

    return pl.pallas_call(
        body,
        grid=(NP // BN,),
        in_specs=[pl.BlockSpec((BN, H), lambda i: (i, 0)),
                  pl.BlockSpec((BN, H), lambda i: (i, 0)),
                  pl.BlockSpec((BN, DEG_W), lambda i: (i, 0)),
                  pl.BlockSpec((BN, DEG_W), lambda i: (i, 0))],
        out_specs=[pl.BlockSpec((BN, H), lambda i: (i, 0)),
                   pl.BlockSpec((BN, DEG_W), lambda i: (i, 0))],
        out_shape=[jax.ShapeDtypeStruct((NP, H), jnp.float32),
                   jax.ShapeDtypeStruct((NP, DEG_W), jnp.float32)],
    )(P0, P1, D0, D1)


def _final(h, m1, Q0, Q1, inv, WcT, b_out2, NP, H, O, BN):
    def body(h_ref, m1_ref, q0, q1, inv_ref, w_ref, b_ref, o_ref):
        m2 = (q0[...] + q1[...]) * inv_ref[...][:, 0:1]
        comb = jnp.concatenate([h_ref[...], m1_ref[...], m2], axis=1)
        o_ref[...] = jnp.dot(comb, w_ref[...],
                             preferred_element_type=jnp.float32,
                             precision=lax.Precision.HIGHEST) + b_ref[...]

    return pl.pallas_call(
        body,
        grid=(NP // BN,),
        in_specs=[pl.BlockSpec((BN, H), lambda i: (i, 0)),
                  pl.BlockSpec((BN, H), lambda i: (i, 0)),
                  pl.BlockSpec((BN, H), lambda i: (i, 0)),
                  pl.BlockSpec((BN, H), lambda i: (i, 0)),
                  pl.BlockSpec((BN, DEG_W), lambda i: (i, 0)),
                  pl.BlockSpec((3 * H, O), lambda i: (0, 0)),
                  pl.BlockSpec((1, O), lambda i: (0, 0))],
        out_specs=pl.BlockSpec((BN, O), lambda i: (i, 0)),
        out_shape=jax.ShapeDtypeStruct((NP, O), jnp.float32),
    )(h, m1, Q0, Q1, inv, WcT, b_out2)


def kernel(x, edge_index, W_in, b_in, W_out, b_out):
    N, D = x.shape
    H = W_in.shape[0]
    O = W_out.shape[0]
    E = edge_index.shape[1]

    NP = -(-(N + 1) // 2048) * 2048
    GRP = NC * NS * CH * NBUF
    EPW = (-(-E // GRP)) * CH * NBUF
    Epad = EPW * NC * NS
    BN = 1024

    row = edge_index[0].astype(jnp.int32)
    col = edge_index[1].astype(jnp.int32)
    row = jnp.concatenate([row, jnp.full((Epad - E,), N, jnp.int32)])
    col = jnp.concatenate([col, jnp.zeros((Epad - E,), jnp.int32)])
    row = row.reshape(Epad // CH, CH)
    col = col.reshape(Epad // CH, CH)

    x_pad = jnp.pad(x, ((0, NP - N), (0, 0)))
    WinT = W_in.T
    b_in2 = b_in.reshape(1, H)
    W0 = W_out[:, 0:H]
    W12 = W_out[:, H:2 * H] + W_out[:, 2 * H:3 * H]
    W34 = W_out[:, 3 * H:4 * H] + W_out[:, 4 * H:5 * H]
    WcT = jnp.concatenate([W0, W12, W34], axis=1).T
    b_out2 = b_out.reshape(1, O)

    h = _lin_in(x_pad, WinT, b_in2, NP, D, H, BN)

    P, Dg = _make_prop(NP, EPW, H, True)(h, col, row)
    m1, inv = _combine_deg(P[0], P[1], Dg[0], Dg[1], NP, H, BN)

    Q = _make_prop(NP, EPW, H, False)(m1, col, row)
    if isinstance(Q, (list, tuple)):
        Q = Q[0]

    out = _final(h, m1, Q[0], Q[1], inv, WcT, b_out2, NP, H, O, BN)
    return out[:N]


# speedup vs baseline: 9.9565x; 1.9112x over previous
import functools

import jax
import jax.numpy as jnp
from jax import lax
from jax.experimental import pallas as pl
from jax.experimental.pallas import tpu as pltpu
from jax.experimental.pallas import tpu_sc as plsc

NC = 2
NS = 16
CH = 128
DEG_W = 16


NBUF = 4


def _make_prop(NP, EPW, H, with_deg, nbuf):
    npc = EPW // CH
    n_groups = npc // nbuf
    rpt = NP // NS
    nz = rpt // CH
    mesh = plsc.VectorSubcoreMesh(
        core_axis_name="c", subcore_axis_name="s",
        num_cores=NC, num_subcores=NS)
    out_type = [jax.ShapeDtypeStruct((NC, NP, H), jnp.float32)]
    scratch = [
        pltpu.VMEM((npc, CH), jnp.int32),
        pltpu.VMEM((npc, CH), jnp.int32),
        pltpu.VMEM_SHARED((NP, H), jnp.float32),
        pltpu.VMEM_SHARED((NP, H), jnp.float32),
    ]
    scratch += [pltpu.VMEM((CH, H), jnp.float32) for _ in range(nbuf)]
    scratch += [pltpu.SemaphoreType.DMA for _ in range(2 * nbuf)]
    if with_deg:
        out_type.append(jax.ShapeDtypeStruct((NC, NP, DEG_W), jnp.float32))
        scratch += [
            pltpu.VMEM((CH, DEG_W), jnp.float32),
            pltpu.VMEM_SHARED((NP, DEG_W), jnp.float32),
        ]
        scratch += [pltpu.SemaphoreType.DMA for _ in range(nbuf)]

    def body(*refs):
        h_hbm, col_hbm, row_hbm = refs[0:3]
        if with_deg:
            out_hbm, dout_hbm = refs[3:5]
            rest = refs[5:]
        else:
            out_hbm = refs[3]
            rest = refs[4:]
        col_s, row_s, acc, hstage = rest[0:4]
        rows_v = rest[4:4 + nbuf]
        gsem = rest[4 + nbuf:4 + 2 * nbuf]
        ssem = rest[4 + 2 * nbuf:4 + 3 * nbuf]
        if with_deg:
            ones_v, dacc = rest[4 + 3 * nbuf:6 + 3 * nbuf]
            dsem = rest[6 + 3 * nbuf:6 + 4 * nbuf]

        cid = lax.axis_index("c")
        sid = lax.axis_index("s")
        wid = sid * NC + cid
        zero16 = jnp.zeros((16,), jnp.float32)
        one16 = jnp.ones((16,), jnp.float32)

        idesc0 = pltpu.async_copy(col_hbm.at[pl.ds(wid * npc, npc)], col_s,
                                  gsem[0])
        idesc1 = pltpu.async_copy(row_hbm.at[pl.ds(wid * npc, npc)], row_s,
                                  gsem[1])
        row0 = sid * rpt
        hdesc = pltpu.async_copy(h_hbm.at[pl.ds(row0, rpt)],
                                 hstage.at[pl.ds(row0, rpt)], ssem[0])

        @pl.loop(0, CH)
        def _zrows(i):
            @pl.loop(0, H // 16)
            def _zlanes(j):
                rows_v[0][i, pl.ds(j * 16, 16)] = zero16

        if with_deg:
            @pl.loop(0, CH)
            def _fill0(i):
                ones_v[i, pl.ds(0, 16)] = zero16

        @pl.loop(0, nz)
        def _zacc(i):
            pltpu.sync_copy(rows_v[0], acc.at[pl.ds(row0 + i * CH, CH)])
            if with_deg:
                pltpu.sync_copy(ones_v, dacc.at[pl.ds(row0 + i * CH, CH)])

        if with_deg:
            @pl.loop(0, CH)
            def _fill1(i):
                ones_v[i, pl.ds(0, 16)] = one16

        idesc0.wait()
        idesc1.wait()
        hdesc.wait()

        plsc.subcore_barrier()

        for b in range(nbuf):
            pltpu.async_copy(hstage.at[col_s.at[b]], rows_v[b], gsem[b])

        @pl.loop(0, n_groups)
        def _edges(g):
            c0 = g * nbuf
            for b in range(nbuf):
                c = c0 + b
                pltpu.make_async_copy(hstage.at[col_s.at[c]], rows_v[b],
                                      gsem[b]).wait()
                pltpu.async_copy(rows_v[b], acc.at[row_s.at[c]], ssem[b],
                                 add=True)
                if with_deg:
                    pltpu.async_copy(ones_v, dacc.at[row_s.at[c]], dsem[b],
                                     add=True)
            for b in range(nbuf):
                c = c0 + b
                pltpu.make_async_copy(rows_v[b], acc.at[row_s.at[c]],
                                      ssem[b]).wait()
                if with_deg:
                    pltpu.make_async_copy(ones_v, dacc.at[row_s.at[c]],
                                          dsem[b]).wait()

                @pl.when(g + 1 < n_groups)
                def _refill():
                    pltpu.async_copy(hstage.at[col_s.at[c + nbuf]], rows_v[b],
                                     gsem[b])

        plsc.subcore_barrier()

        pltpu.sync_copy(acc.at[pl.ds(row0, rpt)],
                        out_hbm.at[cid, pl.ds(row0, rpt)])
        if with_deg:
            pltpu.sync_copy(dacc.at[pl.ds(row0, rpt)],
                            dout_hbm.at[cid, pl.ds(row0, rpt)])

    return pl.kernel(body, out_type=tuple(out_type), mesh=mesh,
                     scratch_types=tuple(scratch),
                     compiler_params=pltpu.CompilerParams(
                         use_tc_tiling_on_sc=False))


def _lin_in(x_pad, WinT, b_in2, NP, D, H, BN):
    def body(x_ref, w_ref, b_ref, o_ref):
        h = jnp.dot(x_ref[...], w_ref[...],
                    preferred_element_type=jnp.float32,
                    precision=lax.Precision.HIGHEST)
        o_ref[...] = jnp.maximum(h + b_ref[...], 0.0)

    return pl.pallas_call(
        body,
        grid=(NP // BN,),
        in_specs=[pl.BlockSpec((BN, D), lambda i: (i, 0)),
                  pl.BlockSpec((D, H), lambda i: (0, 0)),
                  pl.BlockSpec((1, H), lambda i: (0, 0))],
        out_specs=pl.BlockSpec((BN, H), lambda i: (i, 0)),
        out_shape=jax.ShapeDtypeStruct((NP, H), jnp.float32),
    )(x_pad, WinT, b_in2)


def _combine_deg(P0, P1, D0, D1, NP, H, BN):
    def body(p0, p1, d0, d1, m_ref, inv_ref):
        deg = d0[...] + d1[...]
        inv = jnp.where(deg > 0.0, 1.0 / deg, 0.0)
        inv_ref[...] = inv
        m_ref[...] = (p0[...] + p1[...]) * inv[:, 0:1]

    return pl.pallas_call(
        body,
        grid=(NP // BN,),
        in_specs=[pl.BlockSpec((BN, H), lambda i: (i, 0)),
                  pl.BlockSpec((BN, H), lambda i: (i, 0)),
                  pl.BlockSpec((BN, DEG_W), lambda i: (i, 0)),
                  pl.BlockSpec((BN, DEG_W), lambda i: (i, 0))],
        out_specs=[pl.BlockSpec((BN, H), lambda i: (i, 0)),
                   pl.BlockSpec((BN, DEG_W), lambda i: (i, 0))],
        out_shape=[jax.ShapeDtypeStruct((NP, H), jnp.float32),
                   jax.ShapeDtypeStruct((NP, DEG_W), jnp.float32)],
    )(P0, P1, D0, D1)


def _final(h, m1, Q0, Q1, inv, WcT, b_out2, NP, H, O, BN):
    def body(h_ref, m1_ref, q0, q1, inv_ref, w_ref, b_ref, o_ref):
        m2 = (q0[...] + q1[...]) * inv_ref[...][:, 0:1]
        comb = jnp.concatenate([h_ref[...], m1_ref[...], m2], axis=1)
        o_ref[...] = jnp.dot(comb, w_ref[...],
                             preferred_element_type=jnp.float32,
                             precision=lax.Precision.HIGHEST) + b_ref[...]

    return pl.pallas_call(
        body,
        grid=(NP // BN,),
        in_specs=[pl.BlockSpec((BN, H), lambda i: (i, 0)),
                  pl.BlockSpec((BN, H), lambda i: (i, 0)),
                  pl.BlockSpec((BN, H), lambda i: (i, 0)),
                  pl.BlockSpec((BN, H), lambda i: (i, 0)),
                  pl.BlockSpec((BN, DEG_W), lambda i: (i, 0)),
                  pl.BlockSpec((3 * H, O), lambda i: (0, 0)),
                  pl.BlockSpec((1, O), lambda i: (0, 0))],
        out_specs=pl.BlockSpec((BN, O), lambda i: (i, 0)),
        out_shape=jax.ShapeDtypeStruct((NP, O), jnp.float32),
    )(h, m1, Q0, Q1, inv, WcT, b_out2)


def kernel(x, edge_index, W_in, b_in, W_out, b_out):
    N, D = x.shape
    H = W_in.shape[0]
    O = W_out.shape[0]
    E = edge_index.shape[1]

    NP = -(-(N + 1) // 2048) * 2048
    GRP = NC * NS * CH * NBUF
    EPW = (-(-E // GRP)) * CH * NBUF
    Epad = EPW * NC * NS
    BN = 1024

    row = edge_index[0].astype(jnp.int32)
    col = edge_index[1].astype(jnp.int32)
    row = jnp.concatenate([row, jnp.full((Epad - E,), N, jnp.int32)])
    col = jnp.concatenate([col, jnp.zeros((Epad - E,), jnp.int32)])
    row = row.reshape(Epad // CH, CH)
    col = col.reshape(Epad // CH, CH)

    x_pad = jnp.pad(x, ((0, NP - N), (0, 0)))
    WinT = W_in.T
    b_in2 = b_in.reshape(1, H)
    W0 = W_out[:, 0:H]
    W12 = W_out[:, H:2 * H] + W_out[:, 2 * H:3 * H]
    W34 = W_out[:, 3 * H:4 * H] + W_out[:, 4 * H:5 * H]
    WcT = jnp.concatenate([W0, W12, W34], axis=1).T
    b_out2 = b_out.reshape(1, O)

    h = _lin_in(x_pad, WinT, b_in2, NP, D, H, BN)

    P, Dg = _make_prop(NP, EPW, H, True, 2)(h, col, row)
    m1, inv = _combine_deg(P[0], P[1], Dg[0], Dg[1], NP, H, BN)

    Q = _make_prop(NP, EPW, H, False, 2)(m1, col, row)
    if isinstance(Q, (list, tuple)):
        Q = Q[0]

    out = _final(h, m1, Q[0], Q[1], inv, WcT, b_out2, NP, H, O, BN)
    return out[:N]


# speedup vs baseline: 10.7365x; 1.0783x over previous
import functools

import jax
import jax.numpy as jnp
from jax import lax
from jax.experimental import pallas as pl
from jax.experimental.pallas import tpu as pltpu
from jax.experimental.pallas import tpu_sc as plsc

NC = 2
NS = 16
CH = 128
DEG_W = 16
NBUF = 4


def _make_prop2(NP, EPT, H):
    H2 = H // NC
    npc = EPT // CH
    n_groups = npc // NBUF
    rpt = NP // NS
    nz = rpt // CH
    mesh = plsc.VectorSubcoreMesh(
        core_axis_name="c", subcore_axis_name="s",
        num_cores=NC, num_subcores=NS)
    out_type = (jax.ShapeDtypeStruct((NC, NP, H2), jnp.float32),
                jax.ShapeDtypeStruct((NC, NP, H2), jnp.float32))
    scratch = [
        pltpu.VMEM((npc, CH), jnp.int32),
        pltpu.VMEM((npc, CH), jnp.int32),
        pltpu.VMEM_SHARED((NP, H2), jnp.float32),
        pltpu.VMEM_SHARED((NP, H2), jnp.float32),
        pltpu.VMEM((CH, DEG_W), jnp.float32),
        pltpu.VMEM_SHARED((NP, DEG_W), jnp.float32),
    ]
    scratch += [pltpu.VMEM((CH, H2), jnp.float32) for _ in range(NBUF)]
    scratch += [pltpu.SemaphoreType.DMA for _ in range(3 * NBUF)]

    def body(*refs):
        (h_half, col_hbm, row_hbm, m1_hbm, m2_hbm,
         col_s, row_s, acc, hstage, ones_v, dacc) = refs[0:11]
        rows_v = refs[11:11 + NBUF]
        gsem = refs[11 + NBUF:11 + 2 * NBUF]
        ssem = refs[11 + 2 * NBUF:11 + 3 * NBUF]
        dsem = refs[11 + 3 * NBUF:11 + 4 * NBUF]

        cid = lax.axis_index("c")
        sid = lax.axis_index("s")
        zero16 = jnp.zeros((16,), jnp.float32)
        one16 = jnp.ones((16,), jnp.float32)
        row0 = sid * rpt

        idesc0 = pltpu.async_copy(col_hbm.at[pl.ds(sid * npc, npc)], col_s,
                                  gsem[0])
        idesc1 = pltpu.async_copy(row_hbm.at[pl.ds(sid * npc, npc)], row_s,
                                  gsem[1])
        hdesc = pltpu.async_copy(h_half.at[cid, pl.ds(row0, rpt)],
                                 hstage.at[pl.ds(row0, rpt)], ssem[0])

        def fill_rows0(val16):
            @pl.loop(0, CH)
            def _f(i):
                for j in range(H2 // 16):
                    rows_v[0][i, pl.ds(j * 16, 16)] = val16

        def fill_ones(val16):
            @pl.loop(0, CH)
            def _f(i):
                ones_v[i, pl.ds(0, 16)] = val16

        fill_rows0(zero16)
        fill_ones(zero16)

        @pl.loop(0, nz)
        def _zacc(i):
            pltpu.sync_copy(rows_v[0], acc.at[pl.ds(row0 + i * CH, CH)])
            pltpu.sync_copy(ones_v, dacc.at[pl.ds(row0 + i * CH, CH)])

        fill_ones(one16)
        idesc0.wait()
        idesc1.wait()
        hdesc.wait()

        plsc.subcore_barrier()

        def run_hop(with_deg):
            for b in range(NBUF):
                pltpu.async_copy(hstage.at[col_s.at[b]], rows_v[b], gsem[b])

            @pl.loop(0, n_groups)
            def _edges(g):
                c0 = g * NBUF
                for b in range(NBUF):
                    c = c0 + b
                    pltpu.make_async_copy(hstage.at[col_s.at[c]], rows_v[b],
                                          gsem[b]).wait()
                    pltpu.async_copy(rows_v[b], acc.at[row_s.at[c]], ssem[b],
                                     add=True)
                    if with_deg:
                        pltpu.async_copy(ones_v, dacc.at[row_s.at[c]],
                                         dsem[b], add=True)
                for b in range(NBUF):
                    c = c0 + b
                    pltpu.make_async_copy(rows_v[b], acc.at[row_s.at[c]],
                                          ssem[b]).wait()
                    if with_deg:
                        pltpu.make_async_copy(ones_v, dacc.at[row_s.at[c]],
                                              dsem[b]).wait()

                    @pl.when(g + 1 < n_groups)
                    def _refill():
                        pltpu.async_copy(hstage.at[col_s.at[c + NBUF]],
                                         rows_v[b], gsem[b])

        def combine(out_hbm, first):
            if first:
                fill_rows0(zero16)

            @pl.loop(0, nz)
            def _cmb(k):
                r = row0 + k * CH
                pltpu.sync_copy(acc.at[pl.ds(r, CH)], rows_v[1])
                pltpu.sync_copy(dacc.at[pl.ds(r, CH)], ones_v)

                @pl.loop(0, CH)
                def _rows(i):
                    dv = ones_v[i, pl.ds(0, 16)]
                    inv = jnp.where(dv > 0.0, 1.0 / dv, 0.0)
                    for j in range(H2 // 16):
                        rows_v[1][i, pl.ds(j * 16, 16)] = (
                            rows_v[1][i, pl.ds(j * 16, 16)] * inv)

                pltpu.sync_copy(rows_v[1], out_hbm.at[cid, pl.ds(r, CH)])
                if first:
                    pltpu.sync_copy(rows_v[1], hstage.at[pl.ds(r, CH)])
                    pltpu.sync_copy(rows_v[0], acc.at[pl.ds(r, CH)])

        run_hop(True)
        plsc.subcore_barrier()
        combine(m1_hbm, True)
        plsc.subcore_barrier()
        run_hop(False)
        plsc.subcore_barrier()
        combine(m2_hbm, False)

    return pl.kernel(body, out_type=out_type, mesh=mesh,
                     scratch_types=tuple(scratch),
                     compiler_params=pltpu.CompilerParams(
                         use_tc_tiling_on_sc=False))


def _lin_in(x, WinT, b_in2, N, NP, D, H, BN):
    H2 = H // NC

    def body(x_ref, w_ref, b_ref, o_ref):
        h = jnp.dot(x_ref[...], w_ref[...], preferred_element_type=jnp.float32)
        h = jnp.maximum(h + b_ref[...], 0.0)
        o_ref[0] = h[:, :H2]
        o_ref[1] = h[:, H2:]

    return pl.pallas_call(
        body,
        grid=(NP // BN,),
        in_specs=[pl.BlockSpec((BN, D), lambda i: (i, 0)),
                  pl.BlockSpec((D, H), lambda i: (0, 0)),
                  pl.BlockSpec((1, H), lambda i: (0, 0))],
        out_specs=pl.BlockSpec((NC, BN, H2), lambda i: (0, i, 0)),
        out_shape=jax.ShapeDtypeStruct((NC, NP, H2), jnp.float32),
    )(x, WinT, b_in2)


def _final(h_half, m1, m2, W_out, b_out2, N, NP, H, O, BN):
    H2 = H // NC

    def body(h0, h1, a0, a1, b0, b1, w_ref, b_ref, o_ref):
        comb = jnp.concatenate(
            [h0[0], h1[0], a0[0], a1[0], b0[0], b1[0]], axis=1)
        w = w_ref[...]
        wc = jnp.concatenate(
            [w[:, 0:H],
             w[:, H:2 * H] + w[:, 2 * H:3 * H],
             w[:, 3 * H:4 * H] + w[:, 4 * H:5 * H]], axis=1)
        o_ref[...] = lax.dot_general(
            comb, wc, (((1,), (1,)), ((), ())),
            preferred_element_type=jnp.float32) + b_ref[...]

    def half(c):
        return pl.BlockSpec((1, BN, H2), lambda i, c=c: (c, i, 0))

    return pl.pallas_call(
        body,
        grid=(-(-N // BN),),
        in_specs=[half(0), half(1), half(0), half(1), half(0), half(1),
                  pl.BlockSpec((O, 5 * H), lambda i: (0, 0)),
                  pl.BlockSpec((1, O), lambda i: (0, 0))],
        out_specs=pl.BlockSpec((BN, O), lambda i: (i, 0)),
        out_shape=jax.ShapeDtypeStruct((N, O), jnp.float32),
    )(h_half, h_half, m1, m1, m2, m2, W_out, b_out2)


def kernel(x, edge_index, W_in, b_in, W_out, b_out):
    N, D = x.shape
    H = W_in.shape[0]
    O = W_out.shape[0]
    E = edge_index.shape[1]

    NP = -(-(N + 1) // 2048) * 2048
    GRP = NS * CH * NBUF
    EPT = (-(-E // GRP)) * CH * NBUF
    Epad = EPT * NS
    BN = 2048

    row = edge_index[0].astype(jnp.int32)
    col = edge_index[1].astype(jnp.int32)
    if E % CH == 0:
        row = jnp.pad(row.reshape(E // CH, CH), ((0, (Epad - E) // CH), (0, 0)),
                      constant_values=N)
        col = jnp.pad(col.reshape(E // CH, CH), ((0, (Epad - E) // CH), (0, 0)))
    else:
        row = jnp.concatenate([row, jnp.full((Epad - E,), N, jnp.int32)])
        col = jnp.concatenate([col, jnp.zeros((Epad - E,), jnp.int32)])
        row = row.reshape(Epad // CH, CH)
        col = col.reshape(Epad // CH, CH)

    WinT = W_in.T
    b_in2 = b_in.reshape(1, H)
    b_out2 = b_out.reshape(1, O)

    h_half = _lin_in(x, WinT, b_in2, N, NP, D, H, BN)
    m1, m2 = _make_prop2(NP, EPT, H)(h_half, col, row)
    return _final(h_half, m1, m2, W_out, b_out2, N, NP, H, O, BN)


# speedup vs baseline: 10.9713x; 1.0219x over previous
import functools

import jax
import jax.numpy as jnp
from jax import lax
from jax.experimental import pallas as pl
from jax.experimental.pallas import tpu as pltpu
from jax.experimental.pallas import tpu_sc as plsc

NC = 2
NS = 16
CH = 128
DEG_W = 16
NBUF = 4


def _make_prop2(NP, EPT, H):
    H2 = H // NC
    npc = EPT // CH
    n_groups = npc // NBUF
    rpt = NP // NS
    nz = rpt // CH
    mesh = plsc.VectorSubcoreMesh(
        core_axis_name="c", subcore_axis_name="s",
        num_cores=NC, num_subcores=NS)
    out_type = (jax.ShapeDtypeStruct((NC, NP, H2), jnp.float32),
                jax.ShapeDtypeStruct((NC, NP, H2), jnp.float32))
    scratch = [
        pltpu.VMEM((npc, CH), jnp.int32),
        pltpu.VMEM((npc, CH), jnp.int32),
        pltpu.VMEM_SHARED((NP, H2), jnp.float32),
        pltpu.VMEM_SHARED((NP, H2), jnp.float32),
        pltpu.VMEM((CH, DEG_W), jnp.float32),
        pltpu.VMEM((CH, DEG_W), jnp.float32),
        pltpu.VMEM_SHARED((NP, DEG_W), jnp.float32),
    ]
    scratch += [pltpu.VMEM((CH, H2), jnp.float32) for _ in range(NBUF)]
    scratch += [pltpu.SemaphoreType.DMA for _ in range(3 * NBUF)]

    def body(*refs):
        (h_half, col_hbm, row_hbm, m1_hbm, m2_hbm,
         col_s, row_s, acc, hstage, ones_v, dbuf2, dacc) = refs[0:12]
        rows_v = refs[12:12 + NBUF]
        gsem = refs[12 + NBUF:12 + 2 * NBUF]
        ssem = refs[12 + 2 * NBUF:12 + 3 * NBUF]
        dsem = refs[12 + 3 * NBUF:12 + 4 * NBUF]

        cid = lax.axis_index("c")
        sid = lax.axis_index("s")
        zero16 = jnp.zeros((16,), jnp.float32)
        one16 = jnp.ones((16,), jnp.float32)
        row0 = sid * rpt

        idesc0 = pltpu.async_copy(col_hbm.at[pl.ds(sid * npc, npc)], col_s,
                                  gsem[0])
        idesc1 = pltpu.async_copy(row_hbm.at[pl.ds(sid * npc, npc)], row_s,
                                  gsem[1])
        hdesc = pltpu.async_copy(h_half.at[cid, pl.ds(row0, rpt)],
                                 hstage.at[pl.ds(row0, rpt)], ssem[0])

        def fill_rows0(val16):
            @pl.loop(0, CH)
            def _f(i):
                for j in range(H2 // 16):
                    rows_v[0][i, pl.ds(j * 16, 16)] = val16

        def fill_ones(val16):
            @pl.loop(0, CH)
            def _f(i):
                ones_v[i, pl.ds(0, 16)] = val16

        fill_rows0(zero16)
        fill_ones(zero16)

        @pl.loop(0, nz)
        def _zacc(i):
            pltpu.sync_copy(rows_v[0], acc.at[pl.ds(row0 + i * CH, CH)])
            pltpu.sync_copy(ones_v, dacc.at[pl.ds(row0 + i * CH, CH)])

        fill_ones(one16)
        idesc0.wait()
        idesc1.wait()
        hdesc.wait()

        plsc.subcore_barrier()

        def run_hop(with_deg):
            for b in range(NBUF):
                pltpu.async_copy(hstage.at[col_s.at[b]], rows_v[b], gsem[b])

            @pl.loop(0, n_groups)
            def _edges(g):
                c0 = g * NBUF
                for b in range(NBUF):
                    c = c0 + b
                    pltpu.make_async_copy(hstage.at[col_s.at[c]], rows_v[b],
                                          gsem[b]).wait()
                    pltpu.async_copy(rows_v[b], acc.at[row_s.at[c]], ssem[b],
                                     add=True)
                    if with_deg:
                        pltpu.async_copy(ones_v, dacc.at[row_s.at[c]],
                                         dsem[b], add=True)
                for b in range(NBUF):
                    c = c0 + b
                    pltpu.make_async_copy(rows_v[b], acc.at[row_s.at[c]],
                                          ssem[b]).wait()
                    if with_deg:
                        pltpu.make_async_copy(ones_v, dacc.at[row_s.at[c]],
                                              dsem[b]).wait()

                    @pl.when(g + 1 < n_groups)
                    def _refill():
                        pltpu.async_copy(hstage.at[col_s.at[c + NBUF]],
                                         rows_v[b], gsem[b])

        def combine(out_hbm, first):
            if first:
                fill_rows0(zero16)
            fb = [rows_v[2], rows_v[3]]
            db = [ones_v, dbuf2]
            reads = {}
            writes = {}

            def fire_reads(k):
                r = row0 + k * CH
                reads[k] = (
                    pltpu.async_copy(acc.at[pl.ds(r, CH)], fb[k % 2],
                                     gsem[k % 2]),
                    pltpu.async_copy(dacc.at[pl.ds(r, CH)], db[k % 2],
                                     dsem[k % 2]))

            fire_reads(0)
            if nz > 1:
                fire_reads(1)
            for k in range(nz):
                r = row0 + k * CH
                ra, rd = reads.pop(k)
                ra.wait()
                rd.wait()

                @pl.loop(0, CH)
                def _rows(i, k=k):
                    dv = db[k % 2][i, pl.ds(0, 16)]
                    inv = jnp.where(dv > 0.0, 1.0 / dv, 0.0)
                    for j in range(H2 // 16):
                        fb[k % 2][i, pl.ds(j * 16, 16)] = (
                            fb[k % 2][i, pl.ds(j * 16, 16)] * inv)

                ws = [pltpu.async_copy(fb[k % 2],
                                       out_hbm.at[cid, pl.ds(r, CH)],
                                       ssem[0])]
                if first:
                    ws.append(pltpu.async_copy(fb[k % 2],
                                               hstage.at[pl.ds(r, CH)],
                                               ssem[1]))
                    ws.append(pltpu.async_copy(rows_v[0],
                                               acc.at[pl.ds(r, CH)],
                                               ssem[2]))
                writes[k] = ws
                if k + 2 < nz:
                    for w in writes.pop(k):
                        w.wait()
                    fire_reads(k + 2)
            for k in sorted(writes):
                for w in writes[k]:
                    w.wait()

        run_hop(True)
        plsc.subcore_barrier()
        combine(m1_hbm, True)
        plsc.subcore_barrier()
        run_hop(False)
        plsc.subcore_barrier()
        combine(m2_hbm, False)

    return pl.kernel(body, out_type=out_type, mesh=mesh,
                     scratch_types=tuple(scratch),
                     compiler_params=pltpu.CompilerParams(
                         use_tc_tiling_on_sc=False))


def _lin_in(x, WinT, b_in2, N, NP, D, H, BN):
    H2 = H // NC

    def body(x_ref, w_ref, b_ref, o_ref):
        h = jnp.dot(x_ref[...], w_ref[...], preferred_element_type=jnp.float32)
        h = jnp.maximum(h + b_ref[...], 0.0)
        o_ref[0] = h[:, :H2]
        o_ref[1] = h[:, H2:]

    return pl.pallas_call(
        body,
        grid=(NP // BN,),
        in_specs=[pl.BlockSpec((BN, D), lambda i: (i, 0)),
                  pl.BlockSpec((D, H), lambda i: (0, 0)),
                  pl.BlockSpec((1, H), lambda i: (0, 0))],
        out_specs=pl.BlockSpec((NC, BN, H2), lambda i: (0, i, 0)),
        out_shape=jax.ShapeDtypeStruct((NC, NP, H2), jnp.float32),
    )(x, WinT, b_in2)


def _final(h_half, m1, m2, W_out, b_out2, N, NP, H, O, BN):
    H2 = H // NC

    def body(h0, h1, a0, a1, b0, b1, w_ref, b_ref, o_ref):
        comb = jnp.concatenate(
            [h0[0], h1[0], a0[0], a1[0], b0[0], b1[0]], axis=1)
        w = w_ref[...]
        wc = jnp.concatenate(
            [w[:, 0:H],
             w[:, H:2 * H] + w[:, 2 * H:3 * H],
             w[:, 3 * H:4 * H] + w[:, 4 * H:5 * H]], axis=1)
        o_ref[...] = lax.dot_general(
            comb, wc, (((1,), (1,)), ((), ())),
            preferred_element_type=jnp.float32) + b_ref[...]

    def half(c):
        return pl.BlockSpec((1, BN, H2), lambda i, c=c: (c, i, 0))

    return pl.pallas_call(
        body,
        grid=(-(-N // BN),),
        in_specs=[half(0), half(1), half(0), half(1), half(0), half(1),
                  pl.BlockSpec((O, 5 * H), lambda i: (0, 0)),
                  pl.BlockSpec((1, O), lambda i: (0, 0))],
        out_specs=pl.BlockSpec((BN, O), lambda i: (i, 0)),
        out_shape=jax.ShapeDtypeStruct((N, O), jnp.float32),
    )(h_half, h_half, m1, m1, m2, m2, W_out, b_out2)


def kernel(x, edge_index, W_in, b_in, W_out, b_out):
    N, D = x.shape
    H = W_in.shape[0]
    O = W_out.shape[0]
    E = edge_index.shape[1]

    NP = -(-(N + 1) // 2048) * 2048
    GRP = NS * CH * NBUF
    EPT = (-(-E // GRP)) * CH * NBUF
    Epad = EPT * NS
    BN = 2048

    row = edge_index[0].astype(jnp.int32)
    col = edge_index[1].astype(jnp.int32)
    if E % CH == 0:
        row = jnp.pad(row.reshape(E // CH, CH), ((0, (Epad - E) // CH), (0, 0)),
                      constant_values=N)
        col = jnp.pad(col.reshape(E // CH, CH), ((0, (Epad - E) // CH), (0, 0)))
    else:
        row = jnp.concatenate([row, jnp.full((Epad - E,), N, jnp.int32)])
        col = jnp.concatenate([col, jnp.zeros((Epad - E,), jnp.int32)])
        row = row.reshape(Epad // CH, CH)
        col = col.reshape(Epad // CH, CH)

    WinT = W_in.T
    b_in2 = b_in.reshape(1, H)
    b_out2 = b_out.reshape(1, O)

    h_half = _lin_in(x, WinT, b_in2, N, NP, D, H, BN)
    m1, m2 = _make_prop2(NP, EPT, H)(h_half, col, row)
    return _final(h_half, m1, m2, W_out, b_out2, N, NP, H, O, BN)


# speedup vs baseline: 11.7055x; 1.0669x over previous
import functools

import jax
import jax.numpy as jnp
from jax import lax
from jax.experimental import pallas as pl
from jax.experimental.pallas import tpu as pltpu
from jax.experimental.pallas import tpu_sc as plsc

NC = 2
NS = 16
CH = 128
DEG_W = 16
NBUF = 8


def _make_prop2(NP, EPT, H):
    H2 = H // NC
    npc = EPT // CH
    n_groups = npc // NBUF
    rpt = NP // NS
    nz = rpt // CH
    mesh = plsc.VectorSubcoreMesh(
        core_axis_name="c", subcore_axis_name="s",
        num_cores=NC, num_subcores=NS)
    out_type = (jax.ShapeDtypeStruct((NC, NP, H2), jnp.float32),
                jax.ShapeDtypeStruct((NC, NP, H2), jnp.float32))
    scratch = [
        pltpu.VMEM((npc, CH), jnp.int32),
        pltpu.VMEM((npc, CH), jnp.int32),
        pltpu.VMEM_SHARED((NP, H2), jnp.float32),
        pltpu.VMEM_SHARED((NP, H2), jnp.float32),
        pltpu.VMEM((CH, DEG_W), jnp.float32),
        pltpu.VMEM((CH, DEG_W), jnp.float32),
        pltpu.VMEM_SHARED((NP, DEG_W), jnp.float32),
    ]
    scratch += [pltpu.VMEM((CH, H2), jnp.float32) for _ in range(NBUF)]
    scratch += [pltpu.SemaphoreType.DMA for _ in range(3 * NBUF)]

    def body(*refs):
        (h_half, col_hbm, row_hbm, m1_hbm, m2_hbm,
         col_s, row_s, acc, hstage, ones_v, dbuf2, dacc) = refs[0:12]
        rows_v = refs[12:12 + NBUF]
        gsem = refs[12 + NBUF:12 + 2 * NBUF]
        ssem = refs[12 + 2 * NBUF:12 + 3 * NBUF]
        dsem = refs[12 + 3 * NBUF:12 + 4 * NBUF]

        cid = lax.axis_index("c")
        sid = lax.axis_index("s")
        zero16 = jnp.zeros((16,), jnp.float32)
        one16 = jnp.ones((16,), jnp.float32)
        row0 = sid * rpt

        idesc0 = pltpu.async_copy(col_hbm.at[pl.ds(sid * npc, npc)], col_s,
                                  gsem[0])
        idesc1 = pltpu.async_copy(row_hbm.at[pl.ds(sid * npc, npc)], row_s,
                                  gsem[1])
        hdesc = pltpu.async_copy(h_half.at[cid, pl.ds(row0, rpt)],
                                 hstage.at[pl.ds(row0, rpt)], ssem[0])

        def fill_rows0(val16):
            @pl.loop(0, CH)
            def _f(i):
                for j in range(H2 // 16):
                    rows_v[0][i, pl.ds(j * 16, 16)] = val16

        def fill_ones(val16):
            @pl.loop(0, CH)
            def _f(i):
                ones_v[i, pl.ds(0, 16)] = val16

        fill_rows0(zero16)
        fill_ones(zero16)

        @pl.loop(0, nz)
        def _zacc(i):
            pltpu.sync_copy(rows_v[0], acc.at[pl.ds(row0 + i * CH, CH)])
            pltpu.sync_copy(ones_v, dacc.at[pl.ds(row0 + i * CH, CH)])

        fill_ones(one16)
        idesc0.wait()
        idesc1.wait()
        hdesc.wait()

        plsc.subcore_barrier()

        def run_hop(with_deg):
            for b in range(NBUF):
                pltpu.async_copy(hstage.at[col_s.at[b]], rows_v[b], gsem[b])

            @pl.loop(0, n_groups)
            def _edges(g):
                c0 = g * NBUF
                for b in range(NBUF):
                    c = c0 + b
                    pltpu.make_async_copy(hstage.at[col_s.at[c]], rows_v[b],
                                          gsem[b]).wait()
                    pltpu.async_copy(rows_v[b], acc.at[row_s.at[c]], ssem[b],
                                     add=True)
                    if with_deg:
                        pltpu.async_copy(ones_v, dacc.at[row_s.at[c]],
                                         dsem[b], add=True)
                for b in range(NBUF):
                    c = c0 + b
                    pltpu.make_async_copy(rows_v[b], acc.at[row_s.at[c]],
                                          ssem[b]).wait()
                    if with_deg:
                        pltpu.make_async_copy(ones_v, dacc.at[row_s.at[c]],
                                              dsem[b]).wait()

                    @pl.when(g + 1 < n_groups)
                    def _refill():
                        pltpu.async_copy(hstage.at[col_s.at[c + NBUF]],
                                         rows_v[b], gsem[b])

        def combine(out_hbm, first):
            if first:
                fill_rows0(zero16)
            fb = [rows_v[2], rows_v[3]]
            db = [ones_v, dbuf2]
            reads = {}
            writes = {}

            def fire_reads(k):
                r = row0 + k * CH
                reads[k] = (
                    pltpu.async_copy(acc.at[pl.ds(r, CH)], fb[k % 2],
                                     gsem[k % 2]),
                    pltpu.async_copy(dacc.at[pl.ds(r, CH)], db[k % 2],
                                     dsem[k % 2]))

            fire_reads(0)
            if nz > 1:
                fire_reads(1)
            for k in range(nz):
                r = row0 + k * CH
                ra, rd = reads.pop(k)
                ra.wait()
                rd.wait()

                @pl.loop(0, CH)
                def _rows(i, k=k):
                    dv = db[k % 2][i, pl.ds(0, 16)]
                    inv = jnp.where(dv > 0.0, 1.0 / dv, 0.0)
                    for j in range(H2 // 16):
                        fb[k % 2][i, pl.ds(j * 16, 16)] = (
                            fb[k % 2][i, pl.ds(j * 16, 16)] * inv)

                ws = [pltpu.async_copy(fb[k % 2],
                                       out_hbm.at[cid, pl.ds(r, CH)],
                                       ssem[0])]
                if first:
                    ws.append(pltpu.async_copy(fb[k % 2],
                                               hstage.at[pl.ds(r, CH)],
                                               ssem[1]))
                    ws.append(pltpu.async_copy(rows_v[0],
                                               acc.at[pl.ds(r, CH)],
                                               ssem[2]))
                writes[k] = ws
                if k + 2 < nz:
                    for w in writes.pop(k):
                        w.wait()
                    fire_reads(k + 2)
            for k in sorted(writes):
                for w in writes[k]:
                    w.wait()

        run_hop(True)
        plsc.subcore_barrier()
        combine(m1_hbm, True)
        plsc.subcore_barrier()
        run_hop(False)
        plsc.subcore_barrier()
        combine(m2_hbm, False)

    return pl.kernel(body, out_type=out_type, mesh=mesh,
                     scratch_types=tuple(scratch),
                     compiler_params=pltpu.CompilerParams(
                         use_tc_tiling_on_sc=False))


def _lin_in(x, WinT, b_in2, N, NP, D, H, BN):
    H2 = H // NC

    def body(x_ref, w_ref, b_ref, o_ref):
        h = jnp.dot(x_ref[...], w_ref[...], preferred_element_type=jnp.float32)
        h = jnp.maximum(h + b_ref[...], 0.0)
        o_ref[0] = h[:, :H2]
        o_ref[1] = h[:, H2:]

    return pl.pallas_call(
        body,
        grid=(NP // BN,),
        in_specs=[pl.BlockSpec((BN, D), lambda i: (i, 0)),
                  pl.BlockSpec((D, H), lambda i: (0, 0)),
                  pl.BlockSpec((1, H), lambda i: (0, 0))],
        out_specs=pl.BlockSpec((NC, BN, H2), lambda i: (0, i, 0)),
        out_shape=jax.ShapeDtypeStruct((NC, NP, H2), jnp.float32),
    )(x, WinT, b_in2)


def _final(h_half, m1, m2, W_out, b_out2, N, NP, H, O, BN):
    H2 = H // NC

    def body(h0, h1, a0, a1, b0, b1, w_ref, b_ref, o_ref):
        comb = jnp.concatenate(
            [h0[0], h1[0], a0[0], a1[0], b0[0], b1[0]], axis=1)
        w = w_ref[...]
        wc = jnp.concatenate(
            [w[:, 0:H],
             w[:, H:2 * H] + w[:, 2 * H:3 * H],
             w[:, 3 * H:4 * H] + w[:, 4 * H:5 * H]], axis=1)
        o_ref[...] = lax.dot_general(
            comb, wc, (((1,), (1,)), ((), ())),
            preferred_element_type=jnp.float32) + b_ref[...]

    def half(c):
        return pl.BlockSpec((1, BN, H2), lambda i, c=c: (c, i, 0))

    return pl.pallas_call(
        body,
        grid=(-(-N // BN),),
        in_specs=[half(0), half(1), half(0), half(1), half(0), half(1),
                  pl.BlockSpec((O, 5 * H), lambda i: (0, 0)),
                  pl.BlockSpec((1, O), lambda i: (0, 0))],
        out_specs=pl.BlockSpec((BN, O), lambda i: (i, 0)),
        out_shape=jax.ShapeDtypeStruct((N, O), jnp.float32),
    )(h_half, h_half, m1, m1, m2, m2, W_out, b_out2)


def kernel(x, edge_index, W_in, b_in, W_out, b_out):
    N, D = x.shape
    H = W_in.shape[0]
    O = W_out.shape[0]
    E = edge_index.shape[1]

    NP = -(-(N + 1) // 2048) * 2048
    GRP = NS * CH * NBUF
    EPT = (-(-E // GRP)) * CH * NBUF
    Epad = EPT * NS
    BN = 2048

    row = edge_index[0].astype(jnp.int32)
    col = edge_index[1].astype(jnp.int32)
    if E % CH == 0:
        row = jnp.pad(row.reshape(E // CH, CH), ((0, (Epad - E) // CH), (0, 0)),
                      constant_values=N)
        col = jnp.pad(col.reshape(E // CH, CH), ((0, (Epad - E) // CH), (0, 0)))
    else:
        row = jnp.concatenate([row, jnp.full((Epad - E,), N, jnp.int32)])
        col = jnp.concatenate([col, jnp.zeros((Epad - E,), jnp.int32)])
        row = row.reshape(Epad // CH, CH)
        col = col.reshape(Epad // CH, CH)

    WinT = W_in.T
    b_in2 = b_in.reshape(1, H)
    b_out2 = b_out.reshape(1, O)

    h_half = _lin_in(x, WinT, b_in2, N, NP, D, H, BN)
    m1, m2 = _make_prop2(NP, EPT, H)(h_half, col, row)
    return _final(h_half, m1, m2, W_out, b_out2, N, NP, H, O, BN)
